# baseline (speedup 1.0000x reference)
import functools

import jax
import jax.numpy as jnp
from jax import lax
from jax.experimental import pallas as pl
from jax.experimental.pallas import tpu as pltpu
from jax.experimental.pallas import tpu_sc as plsc

_L = 16
_NC = 2
_NS = 16
_NW = _NC * _NS
_N = 1024
_CPW = _N // _NW
_ROWS_X = 2
_COLS_X = 3
_NNZ = 4


def _body(row_hbm, col_hbm, val_hbm, y_hbm, out_hbm,
          row_v, col_v, val_v, y_v, out_v):
    wid = lax.axis_index("s") * _NC + lax.axis_index("c")
    base = wid * _CPW

    pltpu.sync_copy(row_hbm, row_v)
    pltpu.sync_copy(col_hbm, col_v)
    pltpu.sync_copy(val_hbm, val_v)
    for j in range(_COLS_X):
        pltpu.sync_copy(y_hbm.at[j, pl.ds(base, _CPW)], y_v.at[j])

    row = row_v[...]
    col = col_v[...]
    val = val_v[...]
    coeff = [[jnp.float32(0.0)] * _COLS_X for _ in range(_ROWS_X)]
    for k in range(_NNZ):
        rk, ck, vk = row[k], col[k], val[k]
        for i in range(_ROWS_X):
            for j in range(_COLS_X):
                hit = (rk == i) & (ck == j)
                coeff[i][j] = coeff[i][j] + jnp.where(hit, vk, 0.0)

    for i in range(_ROWS_X):
        for g in range(_CPW // _L):
            sl = pl.ds(g * _L, _L)
            acc = coeff[i][0] * y_v[0, sl]
            for j in range(1, _COLS_X):
                acc = acc + coeff[i][j] * y_v[j, sl]
            out_v[i, sl] = acc

    for i in range(_ROWS_X):
        pltpu.sync_copy(out_v.at[i], out_hbm.at[i, pl.ds(base, _CPW)])


@jax.jit
def _spmm(rowp, colp, valp, y):
    mesh = plsc.VectorSubcoreMesh(core_axis_name="c", subcore_axis_name="s")
    return pl.kernel(
        _body,
        mesh=mesh,
        out_type=jax.ShapeDtypeStruct((_ROWS_X, _N), jnp.float32),
        scratch_types=[
            pltpu.VMEM((_L,), jnp.int32),
            pltpu.VMEM((_L,), jnp.int32),
            pltpu.VMEM((_L,), jnp.float32),
            pltpu.VMEM((_COLS_X, _CPW), jnp.float32),
            pltpu.VMEM((_ROWS_X, _CPW), jnp.float32),
        ],
    )(rowp, colp, valp, y)


def kernel(xind, xval, y):
    nnz = xval.shape[0]
    rowp = jnp.zeros((_L,), jnp.int32).at[:nnz].set(xind[0])
    colp = jnp.zeros((_L,), jnp.int32).at[:nnz].set(xind[1])
    valp = jnp.zeros((_L,), jnp.float32).at[:nnz].set(xval)
    return _spmm(rowp, colp, valp, y)


# speedup vs baseline: 1.1251x; 1.1251x over previous
import jax
import jax.numpy as jnp
from jax import lax
from jax.experimental import pallas as pl
from jax.experimental.pallas import tpu as pltpu
from jax.experimental.pallas import tpu_sc as plsc

_L = 16
_NC = 2
_NS = 16
_NW = _NC * _NS
_N = 1024
_CPW = _N // _NW
_ROWS_X = 2
_COLS_X = 3
_NNZ = 4


def _body(coo_hbm, y_hbm, out_hbm, coo_v, y_v, out_v, sem):
    wid = lax.axis_index("s") * _NC + lax.axis_index("c")
    base = wid * _CPW

    copies = [pltpu.async_copy(coo_hbm, coo_v, sem)]
    for j in range(_COLS_X):
        copies.append(
            pltpu.async_copy(y_hbm.at[j, pl.ds(base, _CPW)], y_v.at[j], sem)
        )
    for c in copies:
        c.wait()

    row = coo_v[0, :]
    col = coo_v[1, :]
    val = coo_v[2, :]
    coeff = [[jnp.float32(0.0)] * _COLS_X for _ in range(_ROWS_X)]
    for k in range(_NNZ):
        rk, ck, vk = row[k], col[k], val[k]
        for i in range(_ROWS_X):
            for j in range(_COLS_X):
                hit = (rk == jnp.float32(i)) & (ck == jnp.float32(j))
                coeff[i][j] = coeff[i][j] + jnp.where(hit, vk, 0.0)

    for i in range(_ROWS_X):
        for g in range(_CPW // _L):
            sl = pl.ds(g * _L, _L)
            acc = coeff[i][0] * y_v[0, sl]
            for j in range(1, _COLS_X):
                acc = acc + coeff[i][j] * y_v[j, sl]
            out_v[i, sl] = acc

    outs = [
        pltpu.async_copy(out_v.at[i], out_hbm.at[i, pl.ds(base, _CPW)], sem)
        for i in range(_ROWS_X)
    ]
    for c in outs:
        c.wait()


@jax.jit
def _spmm(coo, y):
    mesh = plsc.VectorSubcoreMesh(core_axis_name="c", subcore_axis_name="s")
    return pl.kernel(
        _body,
        mesh=mesh,
        out_type=jax.ShapeDtypeStruct((_ROWS_X, _N), jnp.float32),
        scratch_types=[
            pltpu.VMEM((_COLS_X, _L), jnp.float32),
            pltpu.VMEM((_COLS_X, _CPW), jnp.float32),
            pltpu.VMEM((_ROWS_X, _CPW), jnp.float32),
            pltpu.SemaphoreType.DMA,
        ],
    )(coo, y)


def kernel(xind, xval, y):
    nnz = xval.shape[0]
    coo = (
        jnp.zeros((_COLS_X, _L), jnp.float32)
        .at[:2, :nnz].set(xind.astype(jnp.float32))
        .at[2, :nnz].set(xval)
    )
    return _spmm(coo, y)


# speedup vs baseline: 1.1604x; 1.0314x over previous
import jax
import jax.numpy as jnp
from jax import lax
from jax.experimental import pallas as pl
from jax.experimental.pallas import tpu as pltpu
from jax.experimental.pallas import tpu_sc as plsc

_L = 16
_NC = 2
_NS = 16
_NW = _NC * _NS
_N = 1024
_CPW = _N // _NW
_ROWS_X = 2
_COLS_X = 3
_NNZ = 4


def _body(xind_hbm, xval_hbm, y_hbm, out_hbm, xind_v, xval_v, y_v, out_v, sem):
    wid = lax.axis_index("s") * _NC + lax.axis_index("c")
    base = wid * _CPW

    copies = [
        pltpu.async_copy(xind_hbm.at[0], xind_v.at[0, pl.ds(0, _NNZ)], sem),
        pltpu.async_copy(xind_hbm.at[1], xind_v.at[1, pl.ds(0, _NNZ)], sem),
        pltpu.async_copy(xval_hbm, xval_v.at[pl.ds(0, _NNZ)], sem),
    ]
    for j in range(_COLS_X):
        copies.append(
            pltpu.async_copy(y_hbm.at[j, pl.ds(base, _CPW)], y_v.at[j], sem)
        )
    for c in copies:
        c.wait()

    row = xind_v[0, :]
    col = xind_v[1, :]
    val = xval_v[...]
    coeff = [[jnp.float32(0.0)] * _COLS_X for _ in range(_ROWS_X)]
    for k in range(_NNZ):
        rk = row[k]
        ck = col[k]
        vk = val[k]
        for i in range(_ROWS_X):
            for j in range(_COLS_X):
                hit = (rk == i) & (ck == j)
                coeff[i][j] = coeff[i][j] + jnp.where(hit, vk, 0.0)

    for i in range(_ROWS_X):
        for g in range(_CPW // _L):
            sl = pl.ds(g * _L, _L)
            acc = coeff[i][0] * y_v[0, sl]
            for j in range(1, _COLS_X):
                acc = acc + coeff[i][j] * y_v[j, sl]
            out_v[i, sl] = acc

    outs = [
        pltpu.async_copy(out_v.at[i], out_hbm.at[i, pl.ds(base, _CPW)], sem)
        for i in range(_ROWS_X)
    ]
    for c in outs:
        c.wait()


@jax.jit
def _spmm(xind, xval, y):
    mesh = plsc.VectorSubcoreMesh(core_axis_name="c", subcore_axis_name="s")
    return pl.kernel(
        _body,
        mesh=mesh,
        out_type=jax.ShapeDtypeStruct((_ROWS_X, _N), jnp.float32),
        scratch_types=[
            pltpu.VMEM((2, _L), jnp.int32),
            pltpu.VMEM((_L,), jnp.float32),
            pltpu.VMEM((_COLS_X, _CPW), jnp.float32),
            pltpu.VMEM((_ROWS_X, _CPW), jnp.float32),
            pltpu.SemaphoreType.DMA,
        ],
    )(xind, xval, y)


def kernel(xind, xval, y):
    return _spmm(xind, xval, y)


# speedup vs baseline: 1.2593x; 1.0852x over previous
import jax
import jax.numpy as jnp
from jax import lax
from jax.experimental import pallas as pl
from jax.experimental.pallas import tpu as pltpu
from jax.experimental.pallas import tpu_sc as plsc

_L = 16
_NC = 1
_NS = 16
_NW = _NC * _NS
_N = 1024
_CPW = _N // _NW
_ROWS_X = 2
_COLS_X = 3
_NNZ = 4


def _body(xind_hbm, xval_hbm, y_hbm, out_hbm, xind_v, xval_v, y_v, out_v, sem):
    wid = lax.axis_index("s") * _NC + lax.axis_index("c")
    base = wid * _CPW

    copies = [
        pltpu.async_copy(xind_hbm.at[0], xind_v.at[0, pl.ds(0, _NNZ)], sem),
        pltpu.async_copy(xind_hbm.at[1], xind_v.at[1, pl.ds(0, _NNZ)], sem),
        pltpu.async_copy(xval_hbm, xval_v.at[pl.ds(0, _NNZ)], sem),
    ]
    for j in range(_COLS_X):
        copies.append(
            pltpu.async_copy(y_hbm.at[j, pl.ds(base, _CPW)], y_v.at[j], sem)
        )
    for c in copies:
        c.wait()

    row = xind_v[0, :]
    col = xind_v[1, :]
    val = xval_v[...]
    coeff = [[jnp.float32(0.0)] * _COLS_X for _ in range(_ROWS_X)]
    for k in range(_NNZ):
        rk = row[k]
        ck = col[k]
        vk = val[k]
        for i in range(_ROWS_X):
            for j in range(_COLS_X):
                hit = (rk == i) & (ck == j)
                coeff[i][j] = coeff[i][j] + jnp.where(hit, vk, 0.0)

    for i in range(_ROWS_X):
        for g in range(_CPW // _L):
            sl = pl.ds(g * _L, _L)
            acc = coeff[i][0] * y_v[0, sl]
            for j in range(1, _COLS_X):
                acc = acc + coeff[i][j] * y_v[j, sl]
            out_v[i, sl] = acc

    outs = [
        pltpu.async_copy(out_v.at[i], out_hbm.at[i, pl.ds(base, _CPW)], sem)
        for i in range(_ROWS_X)
    ]
    for c in outs:
        c.wait()


@jax.jit
def _spmm(xind, xval, y):
    mesh = plsc.VectorSubcoreMesh(
        core_axis_name="c", subcore_axis_name="s", num_cores=_NC
    )
    return pl.kernel(
        _body,
        mesh=mesh,
        out_type=jax.ShapeDtypeStruct((_ROWS_X, _N), jnp.float32),
        scratch_types=[
            pltpu.VMEM((2, _L), jnp.int32),
            pltpu.VMEM((_L,), jnp.float32),
            pltpu.VMEM((_COLS_X, _CPW), jnp.float32),
            pltpu.VMEM((_ROWS_X, _CPW), jnp.float32),
            pltpu.SemaphoreType.DMA,
        ],
    )(xind, xval, y)


def kernel(xind, xval, y):
    return _spmm(xind, xval, y)
